# Initial kernel scaffold; baseline (speedup 1.0000x reference)
import jax
import jax.numpy as jnp
from jax.experimental import pallas as pl


def kernel(F_0, edge_index, meshfield, md_W0, md_b0, md_Wh, md_bh, md_W9, md_b9, df_W0, df_b0, df_Wh, df_bh, df_W9, df_b9):
    raise NotImplementedError("write your pallas kernel here")


# speedup vs baseline: 7.9219x; 7.9219x over previous
import functools

import jax
import jax.numpy as jnp
from jax import lax
from jax.experimental import pallas as pl
from jax.experimental.pallas import tpu as pltpu
from jax.experimental.pallas import tpu_sc as plsc

N = 50000
E = 800000
HID = 64
QCH = 16
CHUNK = 128
NTILES = 16
NR = 51200
EP = 802816
NCHUNKS = EP // CHUNK
CPT = NCHUNKS // NTILES
RPT = NR // NTILES
IB = 14
NBLK = CPT // IB
BLK = 512
GB = NR // BLK
EPS = 1e-5
DT = 5 * (4.0 / 200)


_SC_MESH = plsc.VectorSubcoreMesh(
    core_axis_name="c", subcore_axis_name="s", num_cores=2, num_subcores=NTILES
)


@functools.partial(
    pl.kernel,
    out_type=[jax.ShapeDtypeStruct((NR, QCH), jnp.float32)] * 4,
    mesh=_SC_MESH,
    compiler_params=pltpu.CompilerParams(use_tc_tiling_on_sc=False),
    scratch_types=[
        pltpu.VMEM((2, CHUNK, QCH), jnp.float32),
        pltpu.VMEM((2, IB, 2, CHUNK), jnp.int32),
        pltpu.VMEM_SHARED((NR, QCH), jnp.float32),
        pltpu.SemaphoreType.DMA((2,)),
        pltpu.SemaphoreType.DMA((2,)),
        pltpu.SemaphoreType.DMA((2,)),
    ],
)
def _sc_prop(g0, g1, g2, g3, e2, o0, o1, o2, o3,
             rows, idx, acc, sem_g, sem_s, sem_i):
    c = lax.axis_index("c")
    s = lax.axis_index("s")
    base = s * CPT

    def start_iload(B, sl):
        pltpu.async_copy(
            e2.at[pl.ds(base + B * IB, IB)], idx.at[sl], sem_i.at[sl]
        )

    def wait_iload(sl):
        pltpu.make_async_copy(
            e2.at[pl.ds(0, IB)], idx.at[sl], sem_i.at[sl]
        ).wait()

    for phase in range(2):
        ga, gb = ((g0, g2), (g1, g3))[phase]
        oa, ob = ((o0, o2), (o1, o3))[phase]

        zero = jnp.zeros((16,), jnp.float32)
        for i in range(CHUNK):
            rows[0, i, pl.ds(0, 16)] = zero
        for k in range(RPT // CHUNK):
            pltpu.sync_copy(rows.at[0], acc.at[pl.ds(s * RPT + k * CHUNK, CHUNK)])
        start_iload(0, 0)
        start_iload(1, 1)
        plsc.subcore_barrier()

        def start_gather(sl, k, b):
            @pl.when(c == 0)
            def _():
                pltpu.async_copy(ga.at[idx.at[sl, k, 0]], rows.at[b], sem_g.at[b])

            @pl.when(c != 0)
            def _():
                pltpu.async_copy(gb.at[idx.at[sl, k, 0]], rows.at[b], sem_g.at[b])

        def wait_gather(b):
            pltpu.make_async_copy(
                ga.at[idx.at[0, 0, 0]], rows.at[b], sem_g.at[b]
            ).wait()

        def start_scatter(sl, k, b):
            pltpu.async_copy(rows.at[b], acc.at[idx.at[sl, k, 1]], sem_s.at[b],
                             add=True)

        def wait_scatter(b):
            pltpu.make_async_copy(
                rows.at[b], acc.at[idx.at[0, 0, 1]], sem_s.at[b]
            ).wait()

        def block(B, sl):
            wait_iload(sl)
            start_gather(sl, 0, 0)
            for kk in range(IB // 2):
                j0 = 2 * kk
                if kk > 0:
                    wait_scatter(1)
                start_gather(sl, j0 + 1, 1)
                wait_gather(0)
                start_scatter(sl, j0, 0)
                wait_gather(1)
                start_scatter(sl, j0 + 1, 1)
                wait_scatter(0)
                if kk + 1 < IB // 2:
                    start_gather(sl, j0 + 2, 0)
            wait_scatter(1)

            @pl.when(B + 2 < NBLK)
            def _():
                start_iload(B + 2, sl)

        def pair(p, carry):
            block(2 * p, 0)
            block(2 * p + 1, 1)
            return carry

        lax.fori_loop(0, NBLK // 2, pair, 0)
        plsc.subcore_barrier()

        nrb = RPT // CHUNK

        def rb_start(k, b):
            pltpu.async_copy(
                acc.at[pl.ds(s * RPT + k * CHUNK, CHUNK)], rows.at[b],
                sem_g.at[b]
            )

        def rb_flush(k, b):
            pltpu.make_async_copy(
                acc.at[pl.ds(0, CHUNK)], rows.at[b], sem_g.at[b]
            ).wait()
            dst = pl.ds(s * RPT + k * CHUNK, CHUNK)

            @pl.when(c == 0)
            def _():
                pltpu.async_copy(rows.at[b], oa.at[dst], sem_s.at[b])

            @pl.when(c != 0)
            def _():
                pltpu.async_copy(rows.at[b], ob.at[dst], sem_s.at[b])

        def rb_wait_out(b):
            pltpu.make_async_copy(
                rows.at[b], oa.at[pl.ds(0, CHUNK)], sem_s.at[b]
            ).wait()

        rb_start(0, 0)
        for k in range(nrb):
            b = k % 2
            if k >= 1:
                rb_wait_out(1 - b)
            if k + 1 < nrb:
                rb_start(k + 1, 1 - b)
            rb_flush(k, b)
        rb_wait_out((nrb - 1) % 2)


def _row(w):
    return pl.BlockSpec((BLK, w), lambda i: (i, 0))


def _fixed(a, b):
    return pl.BlockSpec((a, b), lambda i: (0, 0))


def _quarters(gn, orefs):
    for q, oref in enumerate(orefs):
        oref[...] = gn[:, q * QCH:(q + 1) * QCH]


def _cat(refs):
    return jnp.concatenate([r[...] for r in refs], axis=1)


_Q4 = [jax.ShapeDtypeStruct((NR, QCH), jnp.float32)] * 4


def _dinv_body(deg_ref, dinv_ref):
    i = pl.program_id(0)
    rowid = lax.broadcasted_iota(jnp.int32, (BLK, 1), 0) + i * BLK
    deg = deg_ref[...][:, :1] + 1.0
    dinv_ref[...] = jnp.where(rowid < N, 1.0 / jnp.sqrt(deg), 0.0)


_dinv_call = pl.pallas_call(
    _dinv_body,
    grid=(GB,),
    in_specs=[_row(QCH)],
    out_specs=_row(1),
    out_shape=jax.ShapeDtypeStruct((NR, 1), jnp.float32),
)


def _mm0_body(x_ref, w_ref, dinv_ref, *orefs):
    gn = dinv_ref[...] * jnp.dot(x_ref[...], w_ref[...],
                                 preferred_element_type=jnp.float32)
    _quarters(gn, orefs)


def _mm0_call(xw):
    return pl.pallas_call(
        _mm0_body,
        grid=(GB,),
        in_specs=[_row(xw), _fixed(xw, HID), _row(1)],
        out_specs=[_row(QCH)] * 4,
        out_shape=_Q4,
    )


def _mid_body(relu, s0, s1, s2, s3, g0, g1, g2, g3, dinv, w, b, *orefs):
    u = dinv[...] * (_cat((s0, s1, s2, s3)) + _cat((g0, g1, g2, g3))) + b[...]
    if relu:
        u = jnp.maximum(u, 0.0)
    gn = dinv[...] * jnp.dot(u, w[...], preferred_element_type=jnp.float32)
    _quarters(gn, orefs)


def _mid_call(relu):
    return pl.pallas_call(
        functools.partial(_mid_body, relu),
        grid=(GB,),
        in_specs=[_row(QCH)] * 8 + [_row(1), _fixed(HID, HID), _fixed(1, HID)],
        out_specs=[_row(QCH)] * 4,
        out_shape=_Q4,
    )


def _ustats_body(s0, s1, s2, s3, g0, g1, g2, g3, dinv, b,
                 u_ref, sum_ref, sq_ref):
    i = pl.program_id(0)
    u = dinv[...] * (_cat((s0, s1, s2, s3)) + _cat((g0, g1, g2, g3))) + b[...]
    u_ref[...] = u

    @pl.when(i == 0)
    def _():
        sum_ref[...] = jnp.zeros_like(sum_ref)
        sq_ref[...] = jnp.zeros_like(sq_ref)

    rowid = lax.broadcasted_iota(jnp.int32, (BLK, 1), 0) + i * BLK
    x = jnp.where(rowid < N, u, 0.0)
    sum_ref[...] += jnp.sum(x, axis=0, keepdims=True)
    sq_ref[...] += jnp.sum(x * x, axis=0, keepdims=True)


_ustats_call = pl.pallas_call(
    _ustats_body,
    grid=(GB,),
    in_specs=[_row(QCH)] * 8 + [_row(1), _fixed(1, HID)],
    out_specs=[_row(HID), _fixed(1, HID), _fixed(1, HID)],
    out_shape=[
        jax.ShapeDtypeStruct((NR, HID), jnp.float32),
        jax.ShapeDtypeStruct((1, HID), jnp.float32),
        jax.ShapeDtypeStruct((1, HID), jnp.float32),
    ],
)


def _inrm_body(u, ssum, ssq, dinv, w, *orefs):
    mu = ssum[...] * (1.0 / N)
    var = ssq[...] * (1.0 / N) - mu * mu
    v = jnp.maximum((u[...] - mu) / jnp.sqrt(var + EPS), 0.0)
    gn = dinv[...] * jnp.dot(v, w[...], preferred_element_type=jnp.float32)
    _quarters(gn, orefs)


_inrm_call = pl.pallas_call(
    _inrm_body,
    grid=(GB,),
    in_specs=[_row(HID), _fixed(1, HID), _fixed(1, HID), _row(1),
              _fixed(HID, HID)],
    out_specs=[_row(QCH)] * 4,
    out_shape=_Q4,
)


def _m2d_body(s0, s1, s2, s3, g0, g1, g2, g3, dinv, b9, f0p, w0a, w0b,
              *orefs):
    mesh = jnp.tanh(
        dinv[...] * (_cat((s0, s1, s2, s3)) + _cat((g0, g1, g2, g3))) + b9[...]
    )
    x = jnp.dot(f0p[...], w0a[...], preferred_element_type=jnp.float32)
    x += jnp.dot(mesh, w0b[...], preferred_element_type=jnp.float32)
    _quarters(dinv[...] * x, orefs)


_m2d_call = pl.pallas_call(
    _m2d_body,
    grid=(GB,),
    in_specs=[_row(QCH)] * 8 + [_row(1), _fixed(1, HID), _row(8),
                                _fixed(8, HID), _fixed(HID, HID)],
    out_specs=[_row(QCH)] * 4,
    out_shape=_Q4,
)


def _fin_body(s0, s1, s2, s3, g0, g1, g2, g3, dinv, b9, f0p, fn_ref, fd_ref):
    fdot = jnp.tanh(
        dinv[...] * (_cat((s0, s1, s2, s3)) + _cat((g0, g1, g2, g3))) + b9[...]
    )
    f064 = jnp.concatenate(
        [f0p[...], jnp.zeros((BLK, HID - 8), jnp.float32)], axis=1
    )
    fn_ref[...] = jnp.tanh(f064 + fdot * DT)
    fd_ref[...] = fdot


_fin_call = pl.pallas_call(
    _fin_body,
    grid=(GB,),
    in_specs=[_row(QCH)] * 8 + [_row(1), _fixed(1, HID), _row(8)],
    out_specs=[_row(HID), _row(HID)],
    out_shape=[
        jax.ShapeDtypeStruct((NR, HID), jnp.float32),
        jax.ShapeDtypeStruct((NR, HID), jnp.float32),
    ],
)


def kernel(F_0, edge_index, meshfield, md_W0, md_b0, md_Wh, md_bh, md_W9,
           md_b9, df_W0, df_b0, df_Wh, df_bh, df_W9, df_b9):
    f32 = jnp.float32

    sent = N + jnp.arange(EP - E, dtype=jnp.int32) % (NR - N)
    rs2 = jnp.concatenate([edge_index[0], sent]).reshape(NCHUNKS, CHUNK)
    cs2 = jnp.concatenate([edge_index[1], sent]).reshape(NCHUNKS, CHUNK)
    e2 = jnp.stack([rs2, cs2], axis=1)

    ones = jnp.ones((NR, QCH), f32)
    deg, _, _, _ = _sc_prop(ones, ones, ones, ones, e2)
    dinv = _dinv_call(deg)

    f0p = jnp.pad(F_0, ((0, NR - N), (0, 4)))
    mfp = jnp.pad(meshfield, ((0, NR - N), (0, 5)))
    md_W0p = jnp.pad(md_W0, ((0, 5), (0, 0)))
    df_W0a = jnp.pad(df_W0[:4], ((0, 4), (0, 0)))
    df_W0b = df_W0[4:]
    df_W9p = jnp.pad(df_W9, ((0, 0), (0, HID - 4)))
    df_b9p = jnp.pad(df_b9, (0, HID - 4))

    def block(g, b0, Wh, bh, W9):
        s = _sc_prop(*g, e2)
        u, ssum, ssq = _ustats_call(*s, *g, dinv, b0.reshape(1, HID))
        g = _inrm_call(u, ssum, ssq, dinv, Wh[0])
        s = _sc_prop(*g, e2)

        def body(carry, wb):
            s, g = carry
            w, bi = wb
            g = _mid_call(True)(*s, *g, dinv, w, bi.reshape(1, HID))
            s = _sc_prop(*g, e2)
            return (tuple(s), tuple(g)), None

        (s, g), _ = lax.scan(body, (tuple(s), tuple(g)), (Wh[1:8], bh[0:7]))
        g = _mid_call(False)(*s, *g, dinv, W9, bh[7].reshape(1, HID))
        s = _sc_prop(*g, e2)
        return s, g

    g = _mm0_call(8)(mfp, md_W0p, dinv)
    s, g = block(g, md_b0, md_Wh, md_bh, md_W9)
    g = _m2d_call(*s, *g, dinv, md_b9.reshape(1, HID), f0p, df_W0a, df_W0b)

    s, g = block(g, df_b0, df_Wh, df_bh, df_W9p)
    fn, fd = _fin_call(*s, *g, dinv, df_b9p.reshape(1, HID), f0p)

    return (fn[:N, :4][:, None, :], fd[:N, :4][:, None, :])


# speedup vs baseline: 10.7207x; 1.3533x over previous
import functools

import jax
import jax.numpy as jnp
from jax import lax
from jax.experimental import pallas as pl
from jax.experimental.pallas import tpu as pltpu
from jax.experimental.pallas import tpu_sc as plsc

N = 50000
E = 800000
HID = 64
QCH = 16
CHUNK = 128
NTILES = 16
NR = 51200
EP = 802816
NCHUNKS = EP // CHUNK
CPT = NCHUNKS // NTILES
RPT = NR // NTILES
IB = 28
NBLK = CPT // IB
NBUF = 8
BLK = 512
GB = NR // BLK
EPS = 1e-5
DT = 5 * (4.0 / 200)


_SC_MESH = plsc.VectorSubcoreMesh(
    core_axis_name="c", subcore_axis_name="s", num_cores=2, num_subcores=NTILES
)


@functools.partial(
    pl.kernel,
    out_type=[jax.ShapeDtypeStruct((NR, QCH), jnp.float32)] * 4,
    mesh=_SC_MESH,
    compiler_params=pltpu.CompilerParams(use_tc_tiling_on_sc=False),
    scratch_types=[
        pltpu.VMEM((NBUF, CHUNK, QCH), jnp.float32),
        pltpu.VMEM((2, IB, 2, CHUNK), jnp.int32),
        pltpu.VMEM_SHARED((NR, QCH), jnp.float32),
        pltpu.SemaphoreType.DMA((NBUF,)),
        pltpu.SemaphoreType.DMA((NBUF,)),
        pltpu.SemaphoreType.DMA((2,)),
    ],
)
def _sc_prop(g0, g1, g2, g3, e2, o0, o1, o2, o3,
             rows, idx, acc, sem_g, sem_s, sem_i):
    c = lax.axis_index("c")
    s = lax.axis_index("s")
    base = s * CPT

    def start_iload(B, sl):
        pltpu.async_copy(
            e2.at[pl.ds(base + B * IB, IB)], idx.at[sl], sem_i.at[sl]
        )

    def wait_iload(sl):
        pltpu.make_async_copy(
            e2.at[pl.ds(0, IB)], idx.at[sl], sem_i.at[sl]
        ).wait()

    for phase in range(2):
        ga, gb = ((g0, g2), (g1, g3))[phase]
        oa, ob = ((o0, o2), (o1, o3))[phase]

        zero = jnp.zeros((16,), jnp.float32)
        for i in range(CHUNK):
            rows[0, i, pl.ds(0, 16)] = zero
        nrb = RPT // CHUNK
        for k in range(nrb):
            pltpu.async_copy(
                rows.at[0], acc.at[pl.ds(s * RPT + k * CHUNK, CHUNK)],
                sem_s.at[k % NBUF]
            )
        for k in range(nrb):
            pltpu.make_async_copy(
                rows.at[0], acc.at[pl.ds(0, CHUNK)], sem_s.at[k % NBUF]
            ).wait()
        start_iload(0, 0)
        start_iload(1, 1)
        plsc.subcore_barrier()

        def start_gather(sl, k, b):
            @pl.when(c == 0)
            def _():
                pltpu.async_copy(ga.at[idx.at[sl, k, 0]], rows.at[b], sem_g.at[b])

            @pl.when(c != 0)
            def _():
                pltpu.async_copy(gb.at[idx.at[sl, k, 0]], rows.at[b], sem_g.at[b])

        def wait_gather(b):
            pltpu.make_async_copy(
                ga.at[idx.at[0, 0, 0]], rows.at[b], sem_g.at[b]
            ).wait()

        def start_scatter(sl, k, b):
            pltpu.async_copy(rows.at[b], acc.at[idx.at[sl, k, 1]], sem_s.at[b],
                             add=True)

        def wait_scatter(b):
            pltpu.make_async_copy(
                rows.at[b], acc.at[idx.at[0, 0, 1]], sem_s.at[b]
            ).wait()

        def block(B, sl):
            wait_iload(sl)
            for k in range(IB):
                b = k % NBUF
                if k >= NBUF:
                    wait_scatter(b)
                start_gather(sl, k, b)
                if k >= 3:
                    jg = k - 3
                    wait_gather(jg % NBUF)
                    start_scatter(sl, jg, jg % NBUF)
            for jg in range(IB - 3, IB):
                wait_gather(jg % NBUF)
                start_scatter(sl, jg, jg % NBUF)
            for jg in range(IB - NBUF, IB):
                wait_scatter(jg % NBUF)

            @pl.when(B + 2 < NBLK)
            def _():
                start_iload(B + 2, sl)

        def pair(p, carry):
            block(2 * p, 0)
            block(2 * p + 1, 1)
            return carry

        lax.fori_loop(0, NBLK // 2, pair, 0)
        plsc.subcore_barrier()


        def rb_start(k, b):
            pltpu.async_copy(
                acc.at[pl.ds(s * RPT + k * CHUNK, CHUNK)], rows.at[b],
                sem_g.at[b]
            )

        def rb_flush(k, b):
            pltpu.make_async_copy(
                acc.at[pl.ds(0, CHUNK)], rows.at[b], sem_g.at[b]
            ).wait()
            dst = pl.ds(s * RPT + k * CHUNK, CHUNK)

            @pl.when(c == 0)
            def _():
                pltpu.async_copy(rows.at[b], oa.at[dst], sem_s.at[b])

            @pl.when(c != 0)
            def _():
                pltpu.async_copy(rows.at[b], ob.at[dst], sem_s.at[b])

        def rb_wait_out(b):
            pltpu.make_async_copy(
                rows.at[b], oa.at[pl.ds(0, CHUNK)], sem_s.at[b]
            ).wait()

        rb_start(0, 0)
        for k in range(nrb):
            b = k % 2
            if k >= 1:
                rb_wait_out(1 - b)
            if k + 1 < nrb:
                rb_start(k + 1, 1 - b)
            rb_flush(k, b)
        rb_wait_out((nrb - 1) % 2)


def _row(w):
    return pl.BlockSpec((BLK, w), lambda i: (i, 0))


def _fixed(a, b):
    return pl.BlockSpec((a, b), lambda i: (0, 0))


def _quarters(gn, orefs):
    for q, oref in enumerate(orefs):
        oref[...] = gn[:, q * QCH:(q + 1) * QCH]


def _cat(refs):
    return jnp.concatenate([r[...] for r in refs], axis=1)


_Q4 = [jax.ShapeDtypeStruct((NR, QCH), jnp.float32)] * 4


def _dinv_body(deg_ref, dinv_ref):
    i = pl.program_id(0)
    rowid = lax.broadcasted_iota(jnp.int32, (BLK, 1), 0) + i * BLK
    deg = deg_ref[...][:, :1] + 1.0
    dinv_ref[...] = jnp.where(rowid < N, 1.0 / jnp.sqrt(deg), 0.0)


_dinv_call = pl.pallas_call(
    _dinv_body,
    grid=(GB,),
    in_specs=[_row(QCH)],
    out_specs=_row(1),
    out_shape=jax.ShapeDtypeStruct((NR, 1), jnp.float32),
)


def _mm0_body(x_ref, w_ref, dinv_ref, *orefs):
    gn = dinv_ref[...] * jnp.dot(x_ref[...], w_ref[...],
                                 preferred_element_type=jnp.float32)
    _quarters(gn, orefs)


def _mm0_call(xw):
    return pl.pallas_call(
        _mm0_body,
        grid=(GB,),
        in_specs=[_row(xw), _fixed(xw, HID), _row(1)],
        out_specs=[_row(QCH)] * 4,
        out_shape=_Q4,
    )


def _mid_body(relu, s0, s1, s2, s3, g0, g1, g2, g3, dinv, w, b, *orefs):
    u = dinv[...] * (_cat((s0, s1, s2, s3)) + _cat((g0, g1, g2, g3))) + b[...]
    if relu:
        u = jnp.maximum(u, 0.0)
    gn = dinv[...] * jnp.dot(u, w[...], preferred_element_type=jnp.float32)
    _quarters(gn, orefs)


def _mid_call(relu):
    return pl.pallas_call(
        functools.partial(_mid_body, relu),
        grid=(GB,),
        in_specs=[_row(QCH)] * 8 + [_row(1), _fixed(HID, HID), _fixed(1, HID)],
        out_specs=[_row(QCH)] * 4,
        out_shape=_Q4,
    )


def _ustats_body(s0, s1, s2, s3, g0, g1, g2, g3, dinv, b,
                 u_ref, sum_ref, sq_ref):
    i = pl.program_id(0)
    u = dinv[...] * (_cat((s0, s1, s2, s3)) + _cat((g0, g1, g2, g3))) + b[...]
    u_ref[...] = u

    @pl.when(i == 0)
    def _():
        sum_ref[...] = jnp.zeros_like(sum_ref)
        sq_ref[...] = jnp.zeros_like(sq_ref)

    rowid = lax.broadcasted_iota(jnp.int32, (BLK, 1), 0) + i * BLK
    x = jnp.where(rowid < N, u, 0.0)
    sum_ref[...] += jnp.sum(x, axis=0, keepdims=True)
    sq_ref[...] += jnp.sum(x * x, axis=0, keepdims=True)


_ustats_call = pl.pallas_call(
    _ustats_body,
    grid=(GB,),
    in_specs=[_row(QCH)] * 8 + [_row(1), _fixed(1, HID)],
    out_specs=[_row(HID), _fixed(1, HID), _fixed(1, HID)],
    out_shape=[
        jax.ShapeDtypeStruct((NR, HID), jnp.float32),
        jax.ShapeDtypeStruct((1, HID), jnp.float32),
        jax.ShapeDtypeStruct((1, HID), jnp.float32),
    ],
)


def _inrm_body(u, ssum, ssq, dinv, w, *orefs):
    mu = ssum[...] * (1.0 / N)
    var = ssq[...] * (1.0 / N) - mu * mu
    v = jnp.maximum((u[...] - mu) / jnp.sqrt(var + EPS), 0.0)
    gn = dinv[...] * jnp.dot(v, w[...], preferred_element_type=jnp.float32)
    _quarters(gn, orefs)


_inrm_call = pl.pallas_call(
    _inrm_body,
    grid=(GB,),
    in_specs=[_row(HID), _fixed(1, HID), _fixed(1, HID), _row(1),
              _fixed(HID, HID)],
    out_specs=[_row(QCH)] * 4,
    out_shape=_Q4,
)


def _m2d_body(s0, s1, s2, s3, g0, g1, g2, g3, dinv, b9, f0p, w0a, w0b,
              *orefs):
    mesh = jnp.tanh(
        dinv[...] * (_cat((s0, s1, s2, s3)) + _cat((g0, g1, g2, g3))) + b9[...]
    )
    x = jnp.dot(f0p[...], w0a[...], preferred_element_type=jnp.float32)
    x += jnp.dot(mesh, w0b[...], preferred_element_type=jnp.float32)
    _quarters(dinv[...] * x, orefs)


_m2d_call = pl.pallas_call(
    _m2d_body,
    grid=(GB,),
    in_specs=[_row(QCH)] * 8 + [_row(1), _fixed(1, HID), _row(8),
                                _fixed(8, HID), _fixed(HID, HID)],
    out_specs=[_row(QCH)] * 4,
    out_shape=_Q4,
)


def _fin_body(s0, s1, s2, s3, g0, g1, g2, g3, dinv, b9, f0p, fn_ref, fd_ref):
    fdot = jnp.tanh(
        dinv[...] * (_cat((s0, s1, s2, s3)) + _cat((g0, g1, g2, g3))) + b9[...]
    )
    f064 = jnp.concatenate(
        [f0p[...], jnp.zeros((BLK, HID - 8), jnp.float32)], axis=1
    )
    fn_ref[...] = jnp.tanh(f064 + fdot * DT)
    fd_ref[...] = fdot


_fin_call = pl.pallas_call(
    _fin_body,
    grid=(GB,),
    in_specs=[_row(QCH)] * 8 + [_row(1), _fixed(1, HID), _row(8)],
    out_specs=[_row(HID), _row(HID)],
    out_shape=[
        jax.ShapeDtypeStruct((NR, HID), jnp.float32),
        jax.ShapeDtypeStruct((NR, HID), jnp.float32),
    ],
)


def kernel(F_0, edge_index, meshfield, md_W0, md_b0, md_Wh, md_bh, md_W9,
           md_b9, df_W0, df_b0, df_Wh, df_bh, df_W9, df_b9):
    f32 = jnp.float32

    sent = N + jnp.arange(EP - E, dtype=jnp.int32) % (NR - N)
    rs2 = jnp.concatenate([edge_index[0], sent]).reshape(NCHUNKS, CHUNK)
    cs2 = jnp.concatenate([edge_index[1], sent]).reshape(NCHUNKS, CHUNK)
    e2 = jnp.stack([rs2, cs2], axis=1)

    ones = jnp.ones((NR, QCH), f32)
    deg, _, _, _ = _sc_prop(ones, ones, ones, ones, e2)
    dinv = _dinv_call(deg)

    f0p = jnp.pad(F_0, ((0, NR - N), (0, 4)))
    mfp = jnp.pad(meshfield, ((0, NR - N), (0, 5)))
    md_W0p = jnp.pad(md_W0, ((0, 5), (0, 0)))
    df_W0a = jnp.pad(df_W0[:4], ((0, 4), (0, 0)))
    df_W0b = df_W0[4:]
    df_W9p = jnp.pad(df_W9, ((0, 0), (0, HID - 4)))
    df_b9p = jnp.pad(df_b9, (0, HID - 4))

    def block(g, b0, Wh, bh, W9):
        s = _sc_prop(*g, e2)
        u, ssum, ssq = _ustats_call(*s, *g, dinv, b0.reshape(1, HID))
        g = _inrm_call(u, ssum, ssq, dinv, Wh[0])
        s = _sc_prop(*g, e2)

        def body(carry, wb):
            s, g = carry
            w, bi = wb
            g = _mid_call(True)(*s, *g, dinv, w, bi.reshape(1, HID))
            s = _sc_prop(*g, e2)
            return (tuple(s), tuple(g)), None

        (s, g), _ = lax.scan(body, (tuple(s), tuple(g)), (Wh[1:8], bh[0:7]))
        g = _mid_call(False)(*s, *g, dinv, W9, bh[7].reshape(1, HID))
        s = _sc_prop(*g, e2)
        return s, g

    g = _mm0_call(8)(mfp, md_W0p, dinv)
    s, g = block(g, md_b0, md_Wh, md_bh, md_W9)
    g = _m2d_call(*s, *g, dinv, md_b9.reshape(1, HID), f0p, df_W0a, df_W0b)

    s, g = block(g, df_b0, df_Wh, df_bh, df_W9p)
    fn, fd = _fin_call(*s, *g, dinv, df_b9p.reshape(1, HID), f0p)

    return (fn[:N, :4][:, None, :], fd[:N, :4][:, None, :])


# speedup vs baseline: 11.4879x; 1.0716x over previous
import functools

import jax
import jax.numpy as jnp
from jax import lax
from jax.experimental import pallas as pl
from jax.experimental.pallas import tpu as pltpu
from jax.experimental.pallas import tpu_sc as plsc

N = 50000
E = 800000
HID = 64
QCH = 16
CHUNK = 128
NTILES = 16
NR = 51200
EP = 802816
NCHUNKS = EP // CHUNK
CPT = NCHUNKS // NTILES
RPT = NR // NTILES
IB = 28
NBLK = CPT // IB
NBUF = 8
BLK = 512
GB = NR // BLK
EPS = 1e-5
DT = 5 * (4.0 / 200)


_SC_MESH = plsc.VectorSubcoreMesh(
    core_axis_name="c", subcore_axis_name="s", num_cores=2, num_subcores=NTILES
)


@functools.partial(
    pl.kernel,
    out_type=[jax.ShapeDtypeStruct((NR, QCH), jnp.float32)] * 4,
    mesh=_SC_MESH,
    compiler_params=pltpu.CompilerParams(use_tc_tiling_on_sc=False),
    scratch_types=[
        pltpu.VMEM((NBUF, CHUNK, QCH), jnp.float32),
        pltpu.VMEM((2, IB, 2, CHUNK), jnp.int32),
        pltpu.VMEM_SHARED((NR, QCH), jnp.float32),
        pltpu.SemaphoreType.DMA((NBUF,)),
        pltpu.SemaphoreType.DMA((NBUF,)),
        pltpu.SemaphoreType.DMA((2,)),
    ],
)
def _sc_prop(g0, g1, g2, g3, e2, o0, o1, o2, o3,
             rows, idx, acc, sem_g, sem_s, sem_i):
    c = lax.axis_index("c")
    s = lax.axis_index("s")
    base = s * CPT

    def start_iload(B, sl):
        pltpu.async_copy(
            e2.at[pl.ds(base + B * IB, IB)], idx.at[sl], sem_i.at[sl]
        )

    def wait_iload(sl):
        pltpu.make_async_copy(
            e2.at[pl.ds(0, IB)], idx.at[sl], sem_i.at[sl]
        ).wait()

    for phase in range(2):
        ga, gb = ((g0, g2), (g1, g3))[phase]
        oa, ob = ((o0, o2), (o1, o3))[phase]

        zero = jnp.zeros((16,), jnp.float32)
        for i in range(CHUNK):
            rows[0, i, pl.ds(0, 16)] = zero
        nrb = RPT // CHUNK
        for k in range(nrb):
            pltpu.async_copy(
                rows.at[0], acc.at[pl.ds(s * RPT + k * CHUNK, CHUNK)],
                sem_s.at[k % NBUF]
            )
        for k in range(nrb):
            pltpu.make_async_copy(
                rows.at[0], acc.at[pl.ds(0, CHUNK)], sem_s.at[k % NBUF]
            ).wait()
        start_iload(0, 0)
        start_iload(1, 1)
        plsc.subcore_barrier()

        def start_gather(sl, k, b):
            @pl.when(c == 0)
            def _():
                pltpu.async_copy(ga.at[idx.at[sl, k, 0]], rows.at[b], sem_g.at[b])

            @pl.when(c != 0)
            def _():
                pltpu.async_copy(gb.at[idx.at[sl, k, 0]], rows.at[b], sem_g.at[b])

        def wait_gather(b):
            pltpu.make_async_copy(
                ga.at[idx.at[0, 0, 0]], rows.at[b], sem_g.at[b]
            ).wait()

        def start_scatter(sl, k, b):
            pltpu.async_copy(rows.at[b], acc.at[idx.at[sl, k, 1]], sem_s.at[b],
                             add=True)

        def wait_scatter(b):
            pltpu.make_async_copy(
                rows.at[b], acc.at[idx.at[0, 0, 1]], sem_s.at[b]
            ).wait()

        def block(B, sl):
            wait_iload(sl)
            for k in range(IB):
                b = k % NBUF
                if k >= NBUF:
                    wait_scatter(b)
                start_gather(sl, k, b)
                if k >= 3:
                    jg = k - 3
                    wait_gather(jg % NBUF)
                    start_scatter(sl, jg, jg % NBUF)
            for jg in range(IB - 3, IB):
                wait_gather(jg % NBUF)
                start_scatter(sl, jg, jg % NBUF)
            for jg in range(IB - NBUF, IB):
                wait_scatter(jg % NBUF)

            @pl.when(B + 2 < NBLK)
            def _():
                start_iload(B + 2, sl)

        def pair(p, carry):
            block(2 * p, 0)
            block(2 * p + 1, 1)
            return carry

        lax.fori_loop(0, NBLK // 2, pair, 0)
        plsc.subcore_barrier()


        def rb_start(k, b):
            pltpu.async_copy(
                acc.at[pl.ds(s * RPT + k * CHUNK, CHUNK)], rows.at[b],
                sem_g.at[b]
            )

        def rb_flush(k, b):
            pltpu.make_async_copy(
                acc.at[pl.ds(0, CHUNK)], rows.at[b], sem_g.at[b]
            ).wait()
            dst = pl.ds(s * RPT + k * CHUNK, CHUNK)

            @pl.when(c == 0)
            def _():
                pltpu.async_copy(rows.at[b], oa.at[dst], sem_s.at[b])

            @pl.when(c != 0)
            def _():
                pltpu.async_copy(rows.at[b], ob.at[dst], sem_s.at[b])

        def rb_wait_out(b):
            pltpu.make_async_copy(
                rows.at[b], oa.at[pl.ds(0, CHUNK)], sem_s.at[b]
            ).wait()

        rb_start(0, 0)
        for k in range(nrb):
            b = k % 2
            if k >= 1:
                rb_wait_out(1 - b)
            if k + 1 < nrb:
                rb_start(k + 1, 1 - b)
            rb_flush(k, b)
        rb_wait_out((nrb - 1) % 2)


CPT1 = NCHUNKS // 32
IB1 = 14
NBLK1 = CPT1 // IB1


@functools.partial(
    pl.kernel,
    out_type=[jax.ShapeDtypeStruct((NR, QCH), jnp.float32)] * 2,
    mesh=_SC_MESH,
    compiler_params=pltpu.CompilerParams(use_tc_tiling_on_sc=False),
    scratch_types=[
        pltpu.VMEM((NBUF, CHUNK, QCH), jnp.float32),
        pltpu.VMEM((2, IB1, 2, CHUNK), jnp.int32),
        pltpu.VMEM_SHARED((NR, QCH), jnp.float32),
        pltpu.SemaphoreType.DMA((NBUF,)),
        pltpu.SemaphoreType.DMA((NBUF,)),
        pltpu.SemaphoreType.DMA((2,)),
    ],
)
def _sc_prop1(g, e2, o0, o1, rows, idx, acc, sem_g, sem_s, sem_i):
    c = lax.axis_index("c")
    s = lax.axis_index("s")
    base = (c * NTILES + s) * CPT1

    def start_iload(B, sl):
        pltpu.async_copy(
            e2.at[pl.ds(base + B * IB1, IB1)], idx.at[sl], sem_i.at[sl]
        )

    def wait_iload(sl):
        pltpu.make_async_copy(
            e2.at[pl.ds(0, IB1)], idx.at[sl], sem_i.at[sl]
        ).wait()

    zero = jnp.zeros((16,), jnp.float32)
    for i in range(CHUNK):
        rows[0, i, pl.ds(0, 16)] = zero
    nrb = RPT // CHUNK
    for k in range(nrb):
        pltpu.async_copy(
            rows.at[0], acc.at[pl.ds(s * RPT + k * CHUNK, CHUNK)],
            sem_s.at[k % NBUF]
        )
    for k in range(nrb):
        pltpu.make_async_copy(
            rows.at[0], acc.at[pl.ds(0, CHUNK)], sem_s.at[k % NBUF]
        ).wait()
    start_iload(0, 0)
    start_iload(1, 1)
    plsc.subcore_barrier()

    def start_gather(sl, k, b):
        pltpu.async_copy(g.at[idx.at[sl, k, 0]], rows.at[b], sem_g.at[b])

    def wait_gather(b):
        pltpu.make_async_copy(
            g.at[idx.at[0, 0, 0]], rows.at[b], sem_g.at[b]
        ).wait()

    def start_scatter(sl, k, b):
        pltpu.async_copy(rows.at[b], acc.at[idx.at[sl, k, 1]], sem_s.at[b],
                         add=True)

    def wait_scatter(b):
        pltpu.make_async_copy(
            rows.at[b], acc.at[idx.at[0, 0, 1]], sem_s.at[b]
        ).wait()

    def block(B, sl):
        wait_iload(sl)
        for k in range(IB1):
            b = k % NBUF
            if k >= NBUF:
                wait_scatter(b)
            start_gather(sl, k, b)
            if k >= 3:
                jg = k - 3
                wait_gather(jg % NBUF)
                start_scatter(sl, jg, jg % NBUF)
        for jg in range(IB1 - 3, IB1):
            wait_gather(jg % NBUF)
            start_scatter(sl, jg, jg % NBUF)
        for jg in range(IB1 - NBUF, IB1):
            wait_scatter(jg % NBUF)

        @pl.when(B + 2 < NBLK1)
        def _():
            start_iload(B + 2, sl)

    def pair(p, carry):
        block(2 * p, 0)
        block(2 * p + 1, 1)
        return carry

    lax.fori_loop(0, NBLK1 // 2, pair, 0)
    plsc.subcore_barrier()

    def rb_start(k, b):
        pltpu.async_copy(
            acc.at[pl.ds(s * RPT + k * CHUNK, CHUNK)], rows.at[b], sem_g.at[b]
        )

    def rb_flush(k, b):
        pltpu.make_async_copy(
            acc.at[pl.ds(0, CHUNK)], rows.at[b], sem_g.at[b]
        ).wait()
        dst = pl.ds(s * RPT + k * CHUNK, CHUNK)

        @pl.when(c == 0)
        def _():
            pltpu.async_copy(rows.at[b], o0.at[dst], sem_s.at[b])

        @pl.when(c != 0)
        def _():
            pltpu.async_copy(rows.at[b], o1.at[dst], sem_s.at[b])

    def rb_wait_out(b):
        pltpu.make_async_copy(
            rows.at[b], o0.at[pl.ds(0, CHUNK)], sem_s.at[b]
        ).wait()

    rb_start(0, 0)
    for k in range(nrb):
        b = k % 2
        if k >= 1:
            rb_wait_out(1 - b)
        if k + 1 < nrb:
            rb_start(k + 1, 1 - b)
        rb_flush(k, b)
    rb_wait_out((nrb - 1) % 2)


def _row(w):
    return pl.BlockSpec((BLK, w), lambda i: (i, 0))


def _fixed(a, b):
    return pl.BlockSpec((a, b), lambda i: (0, 0))


def _quarters(gn, orefs):
    for q, oref in enumerate(orefs):
        oref[...] = gn[:, q * QCH:(q + 1) * QCH]


def _cat(refs):
    return jnp.concatenate([r[...] for r in refs], axis=1)


_Q4 = [jax.ShapeDtypeStruct((NR, QCH), jnp.float32)] * 4


def _dinv_body(d0_ref, d1_ref, mf_ref, dinv_ref, mfq_ref):
    i = pl.program_id(0)
    rowid = lax.broadcasted_iota(jnp.int32, (BLK, 1), 0) + i * BLK
    deg = d0_ref[...][:, :1] + d1_ref[...][:, :1] + 1.0
    dinv = jnp.where(rowid < N, 1.0 / jnp.sqrt(deg), 0.0)
    dinv_ref[...] = dinv
    mfq_ref[...] = dinv * mf_ref[...]


_dinv_call = pl.pallas_call(
    _dinv_body,
    grid=(GB,),
    in_specs=[_row(QCH), _row(QCH), _row(QCH)],
    out_specs=[_row(1), _row(QCH)],
    out_shape=[
        jax.ShapeDtypeStruct((NR, 1), jnp.float32),
        jax.ShapeDtypeStruct((NR, QCH), jnp.float32),
    ],
)


def _ustats0_body(z0, z1, mfq, dinv, w, b, u_ref, sum_ref, sq_ref):
    i = pl.program_id(0)
    z = dinv[...] * (z0[...] + z1[...] + mfq[...])
    u = jnp.dot(z, w[...], preferred_element_type=jnp.float32) + b[...]
    u_ref[...] = u

    @pl.when(i == 0)
    def _():
        sum_ref[...] = jnp.zeros_like(sum_ref)
        sq_ref[...] = jnp.zeros_like(sq_ref)

    rowid = lax.broadcasted_iota(jnp.int32, (BLK, 1), 0) + i * BLK
    x = jnp.where(rowid < N, u, 0.0)
    sum_ref[...] += jnp.sum(x, axis=0, keepdims=True)
    sq_ref[...] += jnp.sum(x * x, axis=0, keepdims=True)


_ustats0_call = pl.pallas_call(
    _ustats0_body,
    grid=(GB,),
    in_specs=[_row(QCH), _row(QCH), _row(QCH), _row(1), _fixed(QCH, HID),
              _fixed(1, HID)],
    out_specs=[_row(HID), _fixed(1, HID), _fixed(1, HID)],
    out_shape=[
        jax.ShapeDtypeStruct((NR, HID), jnp.float32),
        jax.ShapeDtypeStruct((1, HID), jnp.float32),
        jax.ShapeDtypeStruct((1, HID), jnp.float32),
    ],
)


def _midq_body(s0, s1, s2, s3, g0, g1, g2, g3, dinv, w, b, oref):
    u = dinv[...] * (_cat((s0, s1, s2, s3)) + _cat((g0, g1, g2, g3))) + b[...]
    oref[...] = dinv[...] * jnp.dot(u, w[...],
                                    preferred_element_type=jnp.float32)


_midq_call = pl.pallas_call(
    _midq_body,
    grid=(GB,),
    in_specs=[_row(QCH)] * 8 + [_row(1), _fixed(HID, QCH), _fixed(1, HID)],
    out_specs=_row(QCH),
    out_shape=jax.ShapeDtypeStruct((NR, QCH), jnp.float32),
)


def _finq_body(s0, s1, gq, dinv, b9, f0q, fn_ref, fd_ref):
    fdot = jnp.tanh(dinv[...] * (s0[...] + s1[...] + gq[...]) + b9[...])
    fn_ref[...] = jnp.tanh(f0q[...] + fdot * DT)
    fd_ref[...] = fdot


_finq_call = pl.pallas_call(
    _finq_body,
    grid=(GB,),
    in_specs=[_row(QCH), _row(QCH), _row(QCH), _row(1), _fixed(1, QCH),
              _row(QCH)],
    out_specs=[_row(QCH), _row(QCH)],
    out_shape=[
        jax.ShapeDtypeStruct((NR, QCH), jnp.float32),
        jax.ShapeDtypeStruct((NR, QCH), jnp.float32),
    ],
)


def _mid_body(relu, s0, s1, s2, s3, g0, g1, g2, g3, dinv, w, b, *orefs):
    u = dinv[...] * (_cat((s0, s1, s2, s3)) + _cat((g0, g1, g2, g3))) + b[...]
    if relu:
        u = jnp.maximum(u, 0.0)
    gn = dinv[...] * jnp.dot(u, w[...], preferred_element_type=jnp.float32)
    _quarters(gn, orefs)


def _mid_call(relu):
    return pl.pallas_call(
        functools.partial(_mid_body, relu),
        grid=(GB,),
        in_specs=[_row(QCH)] * 8 + [_row(1), _fixed(HID, HID), _fixed(1, HID)],
        out_specs=[_row(QCH)] * 4,
        out_shape=_Q4,
    )


def _ustats_body(s0, s1, s2, s3, g0, g1, g2, g3, dinv, b,
                 u_ref, sum_ref, sq_ref):
    i = pl.program_id(0)
    u = dinv[...] * (_cat((s0, s1, s2, s3)) + _cat((g0, g1, g2, g3))) + b[...]
    u_ref[...] = u

    @pl.when(i == 0)
    def _():
        sum_ref[...] = jnp.zeros_like(sum_ref)
        sq_ref[...] = jnp.zeros_like(sq_ref)

    rowid = lax.broadcasted_iota(jnp.int32, (BLK, 1), 0) + i * BLK
    x = jnp.where(rowid < N, u, 0.0)
    sum_ref[...] += jnp.sum(x, axis=0, keepdims=True)
    sq_ref[...] += jnp.sum(x * x, axis=0, keepdims=True)


_ustats_call = pl.pallas_call(
    _ustats_body,
    grid=(GB,),
    in_specs=[_row(QCH)] * 8 + [_row(1), _fixed(1, HID)],
    out_specs=[_row(HID), _fixed(1, HID), _fixed(1, HID)],
    out_shape=[
        jax.ShapeDtypeStruct((NR, HID), jnp.float32),
        jax.ShapeDtypeStruct((1, HID), jnp.float32),
        jax.ShapeDtypeStruct((1, HID), jnp.float32),
    ],
)


def _inrm_body(u, ssum, ssq, dinv, w, *orefs):
    mu = ssum[...] * (1.0 / N)
    var = ssq[...] * (1.0 / N) - mu * mu
    v = jnp.maximum((u[...] - mu) / jnp.sqrt(var + EPS), 0.0)
    gn = dinv[...] * jnp.dot(v, w[...], preferred_element_type=jnp.float32)
    _quarters(gn, orefs)


_inrm_call = pl.pallas_call(
    _inrm_body,
    grid=(GB,),
    in_specs=[_row(HID), _fixed(1, HID), _fixed(1, HID), _row(1),
              _fixed(HID, HID)],
    out_specs=[_row(QCH)] * 4,
    out_shape=_Q4,
)


def _m2d_body(s0, s1, s2, s3, g0, g1, g2, g3, dinv, b9, f0p, w0a, w0b,
              *orefs):
    mesh = jnp.tanh(
        dinv[...] * (_cat((s0, s1, s2, s3)) + _cat((g0, g1, g2, g3))) + b9[...]
    )
    x = jnp.dot(f0p[...], w0a[...], preferred_element_type=jnp.float32)
    x += jnp.dot(mesh, w0b[...], preferred_element_type=jnp.float32)
    _quarters(dinv[...] * x, orefs)


_m2d_call = pl.pallas_call(
    _m2d_body,
    grid=(GB,),
    in_specs=[_row(QCH)] * 8 + [_row(1), _fixed(1, HID), _row(8),
                                _fixed(8, HID), _fixed(HID, HID)],
    out_specs=[_row(QCH)] * 4,
    out_shape=_Q4,
)


def kernel(F_0, edge_index, meshfield, md_W0, md_b0, md_Wh, md_bh, md_W9,
           md_b9, df_W0, df_b0, df_Wh, df_bh, df_W9, df_b9):
    sent = N + jnp.arange(EP - E, dtype=jnp.int32) % (NR - N)
    rs2 = jnp.concatenate([edge_index[0], sent]).reshape(NCHUNKS, CHUNK)
    cs2 = jnp.concatenate([edge_index[1], sent]).reshape(NCHUNKS, CHUNK)
    e2 = jnp.stack([rs2, cs2], axis=1)

    ones = jnp.ones((NR, QCH), jnp.float32)
    d0, d1 = _sc_prop1(ones, e2)
    mf16 = jnp.pad(meshfield, ((0, NR - N), (0, QCH - 3)))
    dinv, mfq = _dinv_call(d0, d1, mf16)

    f0p = jnp.pad(F_0, ((0, NR - N), (0, 4)))
    f0q = jnp.pad(F_0, ((0, NR - N), (0, QCH - 4)))
    md_W016 = jnp.pad(md_W0, ((0, QCH - 3), (0, 0)))
    df_W0a = jnp.pad(df_W0[:4], ((0, 4), (0, 0)))
    df_W0b = df_W0[4:]
    df_W916 = jnp.pad(df_W9, ((0, 0), (0, QCH - 4)))
    df_b9q = jnp.pad(df_b9, (0, QCH - 4))

    def block_core(u, ssum, ssq, Wh, bh):
        g = _inrm_call(u, ssum, ssq, dinv, Wh[0])
        s = _sc_prop(*g, e2)

        def body(carry, wb):
            s, g = carry
            w, bi = wb
            g = _mid_call(True)(*s, *g, dinv, w, bi.reshape(1, HID))
            s = _sc_prop(*g, e2)
            return (tuple(s), tuple(g)), None

        (s, g), _ = lax.scan(body, (tuple(s), tuple(g)), (Wh[1:8], bh[0:7]))
        return s, g

    z0, z1 = _sc_prop1(mfq, e2)
    u, ssum, ssq = _ustats0_call(z0, z1, mfq, dinv, md_W016,
                                 md_b0.reshape(1, HID))
    s, g = block_core(u, ssum, ssq, md_Wh, md_bh)
    g = _mid_call(False)(*s, *g, dinv, md_W9, md_bh[7].reshape(1, HID))
    s = _sc_prop(*g, e2)
    g = _m2d_call(*s, *g, dinv, md_b9.reshape(1, HID), f0p, df_W0a, df_W0b)

    s = _sc_prop(*g, e2)
    u, ssum, ssq = _ustats_call(*s, *g, dinv, df_b0.reshape(1, HID))
    s, g = block_core(u, ssum, ssq, df_Wh, df_bh)
    gq = _midq_call(*s, *g, dinv, df_W916, df_bh[7].reshape(1, HID))
    s0, s1 = _sc_prop1(gq, e2)
    fn, fd = _finq_call(s0, s1, gq, dinv, df_b9q.reshape(1, QCH), f0q)

    return (fn[:N, :4][:, None, :], fd[:N, :4][:, None, :])
